# baseline (speedup 1.0000x reference)
import jax
import jax.numpy as jnp
import numpy as np
from jax import lax
from jax.experimental import pallas as pl
from jax.experimental.pallas import tpu as pltpu
from jax.experimental.pallas import tpu_sc as plsc

_B = 4096
_L = 200
_NITEMS = 100000
_NUSERS = 10000
_NC = 2
_NS = 16
_NW = _NC * _NS
_PER_W = _B * _L // _NW
_CHUNK = 128
_SUPER = 3200
_NSUPER = _PER_W // _SUPER
_CPS = _SUPER // _CHUNK
_UPW = _B // _NW
_MSEC = _NITEMS + 1
_WSEC = _NITEMS + 1
_RSEC = _NUSERS
_MWROWS = 3 * _MSEC
_WWROWS = 7 * _WSEC
_RWROWS = 13 * _RSEC


def _sc_body(logf, t1f, t2f, ridxf, colpat, mt1, wt1, rt1, out,
             cbuf, t1buf, t2buf, rbuf, colbuf, midx, widx,
             moff, woff, rwidx, mbig, wbig, rbig, obuf, sem):
    wid = lax.axis_index("s") * _NC + lax.axis_index("c")
    base = wid * _PER_W
    iota = lax.iota(jnp.int32, 16)
    iota24 = iota * 24
    pltpu.sync_copy(colpat, colbuf)

    def super_body(s, _):
        soff = base + s * _SUPER
        pltpu.sync_copy(logf.at[pl.ds(soff, _SUPER)], cbuf)
        pltpu.sync_copy(t1f.at[pl.ds(soff, _SUPER)], t1buf)
        pltpu.sync_copy(t2f.at[pl.ds(soff, _SUPER)], t2buf)
        pltpu.sync_copy(ridxf.at[pl.ds(soff, _SUPER)], rbuf)

        def chunk_body(jj, _):
            coff = jj * _CHUNK
            for v in range(_CHUNK // 16):
                sl = pl.ds(coff + v * 16, 16)
                dsl = pl.ds(v * 16, 16)
                c = cbuf[sl]
                t1 = t1buf[sl]
                t2 = t2buf[sl]
                midx[dsl] = lax.shift_right_logical(t1, 3) * _MSEC + c
                moff[dsl] = (t1 & 7) * 16
                widx[dsl] = lax.shift_right_logical(t2, 4) * _WSEC + c
                woff[dsl] = (t2 & 15) * 8
                rwidx[dsl] = rbuf[sl]
            cm = pltpu.async_copy(mt1.at[midx], mbig, sem)
            cw = pltpu.async_copy(wt1.at[widx], wbig, sem)
            cr = pltpu.async_copy(rt1.at[rwidx], rbig, sem)
            cm.wait()
            cw.wait()
            cr.wait()
            for v in range(_CHUNK // 16):
                dsl = pl.ds(v * 16, 16)
                rows = iota + (v * 16)
                mo = moff[dsl]
                wo = woff[dsl]
                rc = colbuf[pl.ds(coff + v * 16, 16)]
                od = iota24 + (v * 384)
                for k in range(12):
                    val = plsc.load_gather(mbig, [rows, mo + k])
                    plsc.store_scatter(obuf, [od + k], val)
                for k in range(6):
                    val = plsc.load_gather(wbig, [rows, wo + k])
                    plsc.store_scatter(obuf, [od + (12 + k)], val)
                for k in range(6):
                    val = plsc.load_gather(rbig, [rows, rc + k])
                    plsc.store_scatter(obuf, [od + (18 + k)], val)
            pltpu.sync_copy(obuf, out.at[pl.ds((soff + coff) * 24,
                                               _CHUNK * 24)])
            return 0

        lax.fori_loop(0, _CPS, chunk_body, 0)
        return 0

    lax.fori_loop(0, _NSUPER, super_body, 0)


def _sc_gather(logf, t1f, t2f, ridxf, colpat, mt1, wt1, rt1):
    mesh = plsc.VectorSubcoreMesh(
        core_axis_name="c", subcore_axis_name="s",
        num_cores=_NC, num_subcores=_NS)
    return pl.kernel(
        _sc_body,
        out_type=jax.ShapeDtypeStruct((_B * _L * 24,), jnp.float32),
        mesh=mesh,
        compiler_params=pltpu.CompilerParams(use_tc_tiling_on_sc=False, needs_layout_passes=False),
        scratch_types=[
            pltpu.VMEM((_SUPER,), jnp.int32),
            pltpu.VMEM((_SUPER,), jnp.int32),
            pltpu.VMEM((_SUPER,), jnp.int32),
            pltpu.VMEM((_SUPER,), jnp.int32),
            pltpu.VMEM((_SUPER,), jnp.int32),
            pltpu.VMEM((_CHUNK,), jnp.int32),
            pltpu.VMEM((_CHUNK,), jnp.int32),
            pltpu.VMEM((_CHUNK,), jnp.int32),
            pltpu.VMEM((_CHUNK,), jnp.int32),
            pltpu.VMEM((_CHUNK,), jnp.int32),
            pltpu.VMEM((_CHUNK, 128), jnp.float32),
            pltpu.VMEM((_CHUNK, 128), jnp.float32),
            pltpu.VMEM((_CHUNK, 128), jnp.float32),
            pltpu.VMEM((_CHUNK * 24,), jnp.float32),
            pltpu.SemaphoreType.DMA,
        ],
    )(logf, t1f, t2f, ridxf, colpat, mt1, wt1, rt1)


def kernel(log_seqs, time1_seqs, time2_seqs, user, month_pop, week_pop, week_eval_pop):
    f32 = month_pop.dtype
    zrow = jnp.zeros((1, 128), f32)
    z4 = jnp.zeros((_NITEMS, 4), f32)
    z8i = jnp.zeros((_NITEMS, 8), f32)
    mtv = month_pop.T
    msecs = []
    for g in range(3):
        pieces = []
        for s in range(8):
            t = 8 * g + s
            pieces += [mtv[:, t * 12:t * 12 + 12], z4]
        msecs += [zrow, jnp.concatenate(pieces, axis=1)]
    mt1 = jnp.concatenate(msecs, axis=0)

    z2 = jnp.zeros((_NITEMS, 2), f32)
    wtv = week_pop.T
    wsecs = []
    for g in range(7):
        pieces = []
        for s in range(16):
            t2 = 16 * g + s
            if t2 == 0 or t2 > 104:
                pieces += [z8i]
            else:
                pieces += [wtv[:, (t2 - 1) * 6:(t2 - 1) * 6 + 6], z2]
        wsecs += [zrow, jnp.concatenate(pieces, axis=1)]
    wt1 = jnp.concatenate(wsecs, axis=0)

    zu8 = jnp.zeros((_NUSERS, 8), f32)
    zu2 = jnp.zeros((_NUSERS, 2), f32)
    wep3 = week_eval_pop.reshape(_NUSERS, 6, _L)
    rsecs = []
    for g in range(13):
        pieces = []
        for s in range(16):
            l = 16 * g + s
            if l >= _L:
                pieces += [zu8]
            else:
                pieces += [wep3[:, :, l], zu2]
        rsecs.append(jnp.concatenate(pieces, axis=1))
    rt1 = jnp.concatenate(rsecs, axis=0)

    ueff = ((user.astype(jnp.int32) + (_NUSERS - 1)) % _NUSERS)
    lrow = (jnp.arange(_L, dtype=jnp.int32) // 16)[None, :]
    ridxf = (lrow * _RSEC + ueff[:, None]).reshape(-1)
    p = np.arange(_SUPER)
    colpat = jnp.asarray(((p % _L) % 16) * 8, dtype=jnp.int32)
    logf = log_seqs.reshape(-1).astype(jnp.int32)
    t1f = time1_seqs.reshape(-1).astype(jnp.int32)
    t2f = time2_seqs.reshape(-1).astype(jnp.int32)
    out = _sc_gather(logf, t1f, t2f, ridxf, colpat, mt1, wt1, rt1)
    return out.reshape(_B, _L, 24)


# speedup vs baseline: 6.4686x; 6.4686x over previous
import jax
import jax.numpy as jnp
from jax import lax
from jax.experimental import pallas as pl
from jax.experimental.pallas import tpu as pltpu
from jax.experimental.pallas import tpu_sc as plsc

_B = 4096
_L = 200
_NITEMS = 100000
_NUSERS = 10000
_NC = 2
_NS = 16
_NW = _NC * _NS
_PER_W = _B * _L // _NW
_CHUNK = 128
_SUPER = 3200
_NSUPER = _PER_W // _SUPER
_CPS = _SUPER // _CHUNK


def _sc_body(logf, t1f, t2f, rqf, mt1, wt1, rt1, out,
             cbuf, t1buf, t2buf, rbuf, mqi, wqi, rqi,
             mvals, wvals, rvals, obuf, sem):
    wid = lax.axis_index("s") * _NC + lax.axis_index("c")
    base = wid * _PER_W
    iota = lax.iota(jnp.int32, 16)
    iota24 = iota * 24
    zero = jnp.zeros((16,), jnp.float32)

    def super_body(s, _):
        soff = base + s * _SUPER
        pltpu.sync_copy(logf.at[pl.ds(soff, _SUPER)], cbuf)
        pltpu.sync_copy(t1f.at[pl.ds(soff, _SUPER)], t1buf)
        pltpu.sync_copy(t2f.at[pl.ds(soff, _SUPER)], t2buf)
        pltpu.sync_copy(rqf.at[pl.ds(soff, _SUPER)], rbuf)

        def chunk_body(jj, _):
            coff = jj * _CHUNK
            for v in range(_CHUNK // 16):
                sl = pl.ds(coff + v * 16, 16)
                c = cbuf[sl]
                qm = jnp.maximum(c * 288 + t1buf[sl] * 12 - 288, 0)
                qw = jnp.maximum(c * 624 + t2buf[sl] * 6 - 630, 0)
                qr = rbuf[sl]
                for k in range(12):
                    mqi[pl.ds(k * _CHUNK + v * 16, 16)] = qm + k
                for k in range(6):
                    wqi[pl.ds(k * _CHUNK + v * 16, 16)] = qw + k
                    rqi[pl.ds(k * _CHUNK + v * 16, 16)] = qr + k
            cps = []
            for k in range(12):
                cps.append(pltpu.async_copy(
                    mt1.at[mqi.at[pl.ds(k * _CHUNK, _CHUNK)]],
                    mvals.at[pl.ds(k * _CHUNK, _CHUNK)], sem))
            for k in range(6):
                cps.append(pltpu.async_copy(
                    wt1.at[wqi.at[pl.ds(k * _CHUNK, _CHUNK)]],
                    wvals.at[pl.ds(k * _CHUNK, _CHUNK)], sem))
                cps.append(pltpu.async_copy(
                    rt1.at[rqi.at[pl.ds(k * _CHUNK, _CHUNK)]],
                    rvals.at[pl.ds(k * _CHUNK, _CHUNK)], sem))
            for cp in cps:
                cp.wait()
            for v in range(_CHUNK // 16):
                sl = pl.ds(coff + v * 16, 16)
                cmask = cbuf[sl] > 0
                wmask = cmask & (t2buf[sl] > 0)
                od = iota24 + (v * 384)
                for k in range(12):
                    val = mvals[pl.ds(k * _CHUNK + v * 16, 16)]
                    val = jnp.where(cmask, val, zero)
                    plsc.store_scatter(obuf, [od + k], val)
                for k in range(6):
                    val = wvals[pl.ds(k * _CHUNK + v * 16, 16)]
                    val = jnp.where(wmask, val, zero)
                    plsc.store_scatter(obuf, [od + (12 + k)], val)
                for k in range(6):
                    val = rvals[pl.ds(k * _CHUNK + v * 16, 16)]
                    plsc.store_scatter(obuf, [od + (18 + k)], val)
            pltpu.sync_copy(obuf, out.at[pl.ds((soff + coff) * 24,
                                               _CHUNK * 24)])
            return 0

        lax.fori_loop(0, _CPS, chunk_body, 0)
        return 0

    lax.fori_loop(0, _NSUPER, super_body, 0)


def _sc_gather(logf, t1f, t2f, rqf, mt1, wt1, rt1):
    mesh = plsc.VectorSubcoreMesh(
        core_axis_name="c", subcore_axis_name="s",
        num_cores=_NC, num_subcores=_NS)
    return pl.kernel(
        _sc_body,
        out_type=jax.ShapeDtypeStruct((_B * _L * 24,), jnp.float32),
        mesh=mesh,
        compiler_params=pltpu.CompilerParams(use_tc_tiling_on_sc=False,
                                             needs_layout_passes=False),
        scratch_types=[
            pltpu.VMEM((_SUPER,), jnp.int32),
            pltpu.VMEM((_SUPER,), jnp.int32),
            pltpu.VMEM((_SUPER,), jnp.int32),
            pltpu.VMEM((_SUPER,), jnp.int32),
            pltpu.VMEM((12 * _CHUNK,), jnp.int32),
            pltpu.VMEM((6 * _CHUNK,), jnp.int32),
            pltpu.VMEM((6 * _CHUNK,), jnp.int32),
            pltpu.VMEM((12 * _CHUNK,), jnp.float32),
            pltpu.VMEM((6 * _CHUNK,), jnp.float32),
            pltpu.VMEM((6 * _CHUNK,), jnp.float32),
            pltpu.VMEM((_CHUNK * 24,), jnp.float32),
            pltpu.SemaphoreType.DMA,
        ],
    )(logf, t1f, t2f, rqf, mt1, wt1, rt1)


def kernel(log_seqs, time1_seqs, time2_seqs, user, month_pop, week_pop, week_eval_pop):
    mt1 = month_pop.T.reshape(-1)
    wt1 = week_pop.T.reshape(-1)
    rt1 = week_eval_pop.T.reshape(-1)
    ueff = (user.astype(jnp.int32) + (_NUSERS - 1)) % _NUSERS
    lcol = jnp.arange(_L, dtype=jnp.int32) * (6 * _NUSERS)
    rqf = (ueff[:, None] * 6 + lcol[None, :]).reshape(-1)
    logf = log_seqs.reshape(-1).astype(jnp.int32)
    t1f = time1_seqs.reshape(-1).astype(jnp.int32)
    t2f = time2_seqs.reshape(-1).astype(jnp.int32)
    out = _sc_gather(logf, t1f, t2f, rqf, mt1, wt1, rt1)
    return out.reshape(_B, _L, 24)


# speedup vs baseline: 7.1294x; 1.1022x over previous
import jax
import jax.numpy as jnp
from jax import lax
from jax.experimental import pallas as pl
from jax.experimental.pallas import tpu as pltpu
from jax.experimental.pallas import tpu_sc as plsc

_B = 4096
_L = 200
_NITEMS = 100000
_NUSERS = 10000
_NC = 2
_NS = 16
_NW = _NC * _NS
_PER_W = _B * _L // _NW
_CHUNK = 128
_SUPER = 2560
_NSUPER = _PER_W // _SUPER
_CPS = _SUPER // _CHUNK


def _sc_body(logf, t1f, t2f, rqf, mt1, wt1, rt1, out,
             cbuf, t1buf, t2buf, rbuf,
             mqiA, wqiA, rqiA, mvalsA, wvalsA, rvalsA, obufA,
             mqiB, wqiB, rqiB, mvalsB, wvalsB, rvalsB, obufB,
             semA, semB):
    wid = lax.axis_index("s") * _NC + lax.axis_index("c")
    base = wid * _PER_W
    iota = lax.iota(jnp.int32, 16)
    iota24 = iota * 24
    zero = jnp.zeros((16,), jnp.float32)

    def idx_and_fire(coff, mqi, wqi, rqi, mvals, wvals, rvals, sem):
        for v in range(_CHUNK // 16):
            sl = pl.ds(coff + v * 16, 16)
            c = cbuf[sl]
            qm = jnp.maximum(c * 288 + t1buf[sl] * 12 - 288, 0)
            qw = jnp.maximum(c * 624 + t2buf[sl] * 6 - 630, 0)
            qr = rbuf[sl]
            for k in range(12):
                mqi[pl.ds(k * _CHUNK + v * 16, 16)] = qm + k
            for k in range(6):
                wqi[pl.ds(k * _CHUNK + v * 16, 16)] = qw + k
                rqi[pl.ds(k * _CHUNK + v * 16, 16)] = qr + k
        cps = []
        for k in range(12):
            cps.append(pltpu.async_copy(
                mt1.at[mqi.at[pl.ds(k * _CHUNK, _CHUNK)]],
                mvals.at[pl.ds(k * _CHUNK, _CHUNK)], sem))
        for k in range(6):
            cps.append(pltpu.async_copy(
                wt1.at[wqi.at[pl.ds(k * _CHUNK, _CHUNK)]],
                wvals.at[pl.ds(k * _CHUNK, _CHUNK)], sem))
            cps.append(pltpu.async_copy(
                rt1.at[rqi.at[pl.ds(k * _CHUNK, _CHUNK)]],
                rvals.at[pl.ds(k * _CHUNK, _CHUNK)], sem))
        return cps

    def assemble_write(soff, coff, mvals, wvals, rvals, obuf):
        for v in range(_CHUNK // 16):
            sl = pl.ds(coff + v * 16, 16)
            cmask = cbuf[sl] > 0
            wmask = cmask & (t2buf[sl] > 0)
            od = iota24 + (v * 384)
            for k in range(12):
                val = mvals[pl.ds(k * _CHUNK + v * 16, 16)]
                val = jnp.where(cmask, val, zero)
                plsc.store_scatter(obuf, [od + k], val)
            for k in range(6):
                val = wvals[pl.ds(k * _CHUNK + v * 16, 16)]
                val = jnp.where(wmask, val, zero)
                plsc.store_scatter(obuf, [od + (12 + k)], val)
            for k in range(6):
                val = rvals[pl.ds(k * _CHUNK + v * 16, 16)]
                plsc.store_scatter(obuf, [od + (18 + k)], val)
        pltpu.sync_copy(obuf, out.at[pl.ds((soff + coff) * 24,
                                           _CHUNK * 24)])

    def super_body(s, _):
        soff = base + s * _SUPER
        pltpu.sync_copy(logf.at[pl.ds(soff, _SUPER)], cbuf)
        pltpu.sync_copy(t1f.at[pl.ds(soff, _SUPER)], t1buf)
        pltpu.sync_copy(t2f.at[pl.ds(soff, _SUPER)], t2buf)
        pltpu.sync_copy(rqf.at[pl.ds(soff, _SUPER)], rbuf)

        def pair_body(jp, _):
            ca = jp * 2 * _CHUNK
            cb = ca + _CHUNK
            cpsA = idx_and_fire(ca, mqiA, wqiA, rqiA,
                                mvalsA, wvalsA, rvalsA, semA)
            cpsB = idx_and_fire(cb, mqiB, wqiB, rqiB,
                                mvalsB, wvalsB, rvalsB, semB)
            for cp in cpsA:
                cp.wait()
            assemble_write(soff, ca, mvalsA, wvalsA, rvalsA, obufA)
            for cp in cpsB:
                cp.wait()
            assemble_write(soff, cb, mvalsB, wvalsB, rvalsB, obufB)
            return 0

        lax.fori_loop(0, _CPS // 2, pair_body, 0)
        return 0

    lax.fori_loop(0, _NSUPER, super_body, 0)


def _sc_gather(logf, t1f, t2f, rqf, mt1, wt1, rt1):
    mesh = plsc.VectorSubcoreMesh(
        core_axis_name="c", subcore_axis_name="s",
        num_cores=_NC, num_subcores=_NS)
    return pl.kernel(
        _sc_body,
        out_type=jax.ShapeDtypeStruct((_B * _L * 24,), jnp.float32),
        mesh=mesh,
        compiler_params=pltpu.CompilerParams(use_tc_tiling_on_sc=False,
                                             needs_layout_passes=False),
        scratch_types=[
            pltpu.VMEM((_SUPER,), jnp.int32),
            pltpu.VMEM((_SUPER,), jnp.int32),
            pltpu.VMEM((_SUPER,), jnp.int32),
            pltpu.VMEM((_SUPER,), jnp.int32),
            pltpu.VMEM((12 * _CHUNK,), jnp.int32),
            pltpu.VMEM((6 * _CHUNK,), jnp.int32),
            pltpu.VMEM((6 * _CHUNK,), jnp.int32),
            pltpu.VMEM((12 * _CHUNK,), jnp.float32),
            pltpu.VMEM((6 * _CHUNK,), jnp.float32),
            pltpu.VMEM((6 * _CHUNK,), jnp.float32),
            pltpu.VMEM((_CHUNK * 24,), jnp.float32),
            pltpu.VMEM((12 * _CHUNK,), jnp.int32),
            pltpu.VMEM((6 * _CHUNK,), jnp.int32),
            pltpu.VMEM((6 * _CHUNK,), jnp.int32),
            pltpu.VMEM((12 * _CHUNK,), jnp.float32),
            pltpu.VMEM((6 * _CHUNK,), jnp.float32),
            pltpu.VMEM((6 * _CHUNK,), jnp.float32),
            pltpu.VMEM((_CHUNK * 24,), jnp.float32),
            pltpu.SemaphoreType.DMA,
            pltpu.SemaphoreType.DMA,
        ],
    )(logf, t1f, t2f, rqf, mt1, wt1, rt1)


def kernel(log_seqs, time1_seqs, time2_seqs, user, month_pop, week_pop, week_eval_pop):
    mt1 = month_pop.T.reshape(-1)
    wt1 = week_pop.T.reshape(-1)
    rt1 = week_eval_pop.T.reshape(-1)
    ueff = (user.astype(jnp.int32) + (_NUSERS - 1)) % _NUSERS
    lcol = jnp.arange(_L, dtype=jnp.int32) * (6 * _NUSERS)
    rqf = (ueff[:, None] * 6 + lcol[None, :]).reshape(-1)
    logf = log_seqs.reshape(-1).astype(jnp.int32)
    t1f = time1_seqs.reshape(-1).astype(jnp.int32)
    t2f = time2_seqs.reshape(-1).astype(jnp.int32)
    out = _sc_gather(logf, t1f, t2f, rqf, mt1, wt1, rt1)
    return out.reshape(_B, _L, 24)


# speedup vs baseline: 7.1988x; 1.0097x over previous
import jax
import jax.numpy as jnp
from jax import lax
from jax.experimental import pallas as pl
from jax.experimental.pallas import tpu as pltpu
from jax.experimental.pallas import tpu_sc as plsc

_B = 4096
_L = 200
_NITEMS = 100000
_NUSERS = 10000
_NC = 2
_NS = 16
_NW = _NC * _NS
_PER_W = _B * _L // _NW
_CHUNK = 128
_SUPER = 2560
_NSUPER = _PER_W // _SUPER
_CPS = _SUPER // _CHUNK


def _sc_body(logf, t1f, t2f, rqf, mt1, wt1, rt1, out,
             cbuf, t1buf, t2buf, rbuf,
             mqiA, wqiA, rqiA, mvalsA, wvalsA, rvalsA, obufA,
             mqiB, wqiB, rqiB, mvalsB, wvalsB, rvalsB, obufB,
             mqiC, wqiC, rqiC, mvalsC, wvalsC, rvalsC, obufC,
             mqiD, wqiD, rqiD, mvalsD, wvalsD, rvalsD, obufD,
             semA, semB, semC, semD):
    wid = lax.axis_index("s") * _NC + lax.axis_index("c")
    base = wid * _PER_W
    iota = lax.iota(jnp.int32, 16)
    iota24 = iota * 24
    zero = jnp.zeros((16,), jnp.float32)

    def idx_and_fire(coff, mqi, wqi, rqi, mvals, wvals, rvals, sem):
        for v in range(_CHUNK // 16):
            sl = pl.ds(coff + v * 16, 16)
            c = cbuf[sl]
            qm = jnp.maximum(c * 288 + t1buf[sl] * 12 - 288, 0)
            qw = jnp.maximum(c * 624 + t2buf[sl] * 6 - 630, 0)
            qr = rbuf[sl]
            for k in range(12):
                mqi[pl.ds(k * _CHUNK + v * 16, 16)] = qm + k
            for k in range(6):
                wqi[pl.ds(k * _CHUNK + v * 16, 16)] = qw + k
                rqi[pl.ds(k * _CHUNK + v * 16, 16)] = qr + k
        cps = []
        for k in range(12):
            cps.append(pltpu.async_copy(
                mt1.at[mqi.at[pl.ds(k * _CHUNK, _CHUNK)]],
                mvals.at[pl.ds(k * _CHUNK, _CHUNK)], sem))
        for k in range(6):
            cps.append(pltpu.async_copy(
                wt1.at[wqi.at[pl.ds(k * _CHUNK, _CHUNK)]],
                wvals.at[pl.ds(k * _CHUNK, _CHUNK)], sem))
            cps.append(pltpu.async_copy(
                rt1.at[rqi.at[pl.ds(k * _CHUNK, _CHUNK)]],
                rvals.at[pl.ds(k * _CHUNK, _CHUNK)], sem))
        return cps

    def assemble_write(soff, coff, mvals, wvals, rvals, obuf):
        for v in range(_CHUNK // 16):
            sl = pl.ds(coff + v * 16, 16)
            cmask = cbuf[sl] > 0
            wmask = cmask & (t2buf[sl] > 0)
            od = iota24 + (v * 384)
            for k in range(12):
                val = mvals[pl.ds(k * _CHUNK + v * 16, 16)]
                val = jnp.where(cmask, val, zero)
                plsc.store_scatter(obuf, [od + k], val)
            for k in range(6):
                val = wvals[pl.ds(k * _CHUNK + v * 16, 16)]
                val = jnp.where(wmask, val, zero)
                plsc.store_scatter(obuf, [od + (12 + k)], val)
            for k in range(6):
                val = rvals[pl.ds(k * _CHUNK + v * 16, 16)]
                plsc.store_scatter(obuf, [od + (18 + k)], val)
        pltpu.sync_copy(obuf, out.at[pl.ds((soff + coff) * 24,
                                           _CHUNK * 24)])

    def super_body(s, _):
        soff = base + s * _SUPER
        pltpu.sync_copy(logf.at[pl.ds(soff, _SUPER)], cbuf)
        pltpu.sync_copy(t1f.at[pl.ds(soff, _SUPER)], t1buf)
        pltpu.sync_copy(t2f.at[pl.ds(soff, _SUPER)], t2buf)
        pltpu.sync_copy(rqf.at[pl.ds(soff, _SUPER)], rbuf)

        sets = [
            (mqiA, wqiA, rqiA, mvalsA, wvalsA, rvalsA, obufA, semA),
            (mqiB, wqiB, rqiB, mvalsB, wvalsB, rvalsB, obufB, semB),
            (mqiC, wqiC, rqiC, mvalsC, wvalsC, rvalsC, obufC, semC),
            (mqiD, wqiD, rqiD, mvalsD, wvalsD, rvalsD, obufD, semD),
        ]

        def quad_body(jp, _):
            base4 = jp * 4 * _CHUNK
            allcps = []
            for n, st in enumerate(sets):
                allcps.append(idx_and_fire(base4 + n * _CHUNK,
                                           st[0], st[1], st[2],
                                           st[3], st[4], st[5], st[7]))
            for n, st in enumerate(sets):
                for cp in allcps[n]:
                    cp.wait()
                assemble_write(soff, base4 + n * _CHUNK,
                               st[3], st[4], st[5], st[6])
            return 0

        lax.fori_loop(0, _CPS // 4, quad_body, 0)
        return 0

    lax.fori_loop(0, _NSUPER, super_body, 0)


def _sc_gather(logf, t1f, t2f, rqf, mt1, wt1, rt1):
    mesh = plsc.VectorSubcoreMesh(
        core_axis_name="c", subcore_axis_name="s",
        num_cores=_NC, num_subcores=_NS)
    return pl.kernel(
        _sc_body,
        out_type=jax.ShapeDtypeStruct((_B * _L * 24,), jnp.float32),
        mesh=mesh,
        compiler_params=pltpu.CompilerParams(use_tc_tiling_on_sc=False,
                                             needs_layout_passes=False),
        scratch_types=[
            pltpu.VMEM((_SUPER,), jnp.int32),
            pltpu.VMEM((_SUPER,), jnp.int32),
            pltpu.VMEM((_SUPER,), jnp.int32),
            pltpu.VMEM((_SUPER,), jnp.int32),
            pltpu.VMEM((12 * _CHUNK,), jnp.int32),
            pltpu.VMEM((6 * _CHUNK,), jnp.int32),
            pltpu.VMEM((6 * _CHUNK,), jnp.int32),
            pltpu.VMEM((12 * _CHUNK,), jnp.float32),
            pltpu.VMEM((6 * _CHUNK,), jnp.float32),
            pltpu.VMEM((6 * _CHUNK,), jnp.float32),
            pltpu.VMEM((_CHUNK * 24,), jnp.float32),
            pltpu.VMEM((12 * _CHUNK,), jnp.int32),
            pltpu.VMEM((6 * _CHUNK,), jnp.int32),
            pltpu.VMEM((6 * _CHUNK,), jnp.int32),
            pltpu.VMEM((12 * _CHUNK,), jnp.float32),
            pltpu.VMEM((6 * _CHUNK,), jnp.float32),
            pltpu.VMEM((6 * _CHUNK,), jnp.float32),
            pltpu.VMEM((_CHUNK * 24,), jnp.float32),
            pltpu.VMEM((12 * _CHUNK,), jnp.int32),
            pltpu.VMEM((6 * _CHUNK,), jnp.int32),
            pltpu.VMEM((6 * _CHUNK,), jnp.int32),
            pltpu.VMEM((12 * _CHUNK,), jnp.float32),
            pltpu.VMEM((6 * _CHUNK,), jnp.float32),
            pltpu.VMEM((6 * _CHUNK,), jnp.float32),
            pltpu.VMEM((_CHUNK * 24,), jnp.float32),
            pltpu.VMEM((12 * _CHUNK,), jnp.int32),
            pltpu.VMEM((6 * _CHUNK,), jnp.int32),
            pltpu.VMEM((6 * _CHUNK,), jnp.int32),
            pltpu.VMEM((12 * _CHUNK,), jnp.float32),
            pltpu.VMEM((6 * _CHUNK,), jnp.float32),
            pltpu.VMEM((6 * _CHUNK,), jnp.float32),
            pltpu.VMEM((_CHUNK * 24,), jnp.float32),
            pltpu.SemaphoreType.DMA,
            pltpu.SemaphoreType.DMA,
            pltpu.SemaphoreType.DMA,
            pltpu.SemaphoreType.DMA,
        ],
    )(logf, t1f, t2f, rqf, mt1, wt1, rt1)


def kernel(log_seqs, time1_seqs, time2_seqs, user, month_pop, week_pop, week_eval_pop):
    mt1 = month_pop.T.reshape(-1)
    wt1 = week_pop.T.reshape(-1)
    rt1 = week_eval_pop.T.reshape(-1)
    ueff = (user.astype(jnp.int32) + (_NUSERS - 1)) % _NUSERS
    lcol = jnp.arange(_L, dtype=jnp.int32) * (6 * _NUSERS)
    rqf = (ueff[:, None] * 6 + lcol[None, :]).reshape(-1)
    logf = log_seqs.reshape(-1).astype(jnp.int32)
    t1f = time1_seqs.reshape(-1).astype(jnp.int32)
    t2f = time2_seqs.reshape(-1).astype(jnp.int32)
    out = _sc_gather(logf, t1f, t2f, rqf, mt1, wt1, rt1)
    return out.reshape(_B, _L, 24)
